# Initial kernel scaffold; baseline (speedup 1.0000x reference)
import jax
import jax.numpy as jnp
from jax.experimental import pallas as pl


def kernel(x, edge_index, W0, b0, W1, b1, Wc):
    raise NotImplementedError("write your pallas kernel here")


# speedup vs baseline: 10.6854x; 10.6854x over previous
import functools

import numpy as np
import jax
import jax.numpy as jnp
from jax import lax
from jax.experimental import pallas as pl
from jax.experimental.pallas import tpu as pltpu
from jax.experimental.pallas import tpu_sc as plsc

N = 10000
E = 320000
D_IN = 128
DH = 64
L = 8
ALPHA = 0.1
THETA = 0.5

NPAD = 10240
NC, NS = 2, 16
NW = NC * NS
B = 128
NB = 80
EPAD = NW * NB * B
EPT = NB * B
RPT = NPAD // NS
RB = 1024
DEGW = 16


@functools.cache
def _sc_kernels():
    mesh = plsc.VectorSubcoreMesh(core_axis_name="c", subcore_axis_name="s")
    params = pltpu.CompilerParams(use_tc_tiling_on_sc=False)

    @functools.partial(
        pl.kernel,
        out_type=jax.ShapeDtypeStruct((NC * NPAD, DEGW), jnp.float32),
        mesh=mesh,
        compiler_params=params,
        scratch_types=[
            pltpu.VMEM((NB, B), jnp.int32),
            pltpu.VMEM((B, DEGW), jnp.float32),
            pltpu.VMEM_SHARED((NPAD, DEGW), jnp.float32),
        ],
    )
    def deg_kernel(dstR, ones2, deg_out, dst_v, ones_v, deg_sh):
        cid = lax.axis_index("c")
        sid = lax.axis_index("s")
        wid = sid * NC + cid
        pltpu.sync_copy(dstR.at[pl.ds(wid * NB, NB)], dst_v)
        pltpu.sync_copy(ones2.at[pl.ds(0, B)], ones_v)
        pltpu.sync_copy(ones2, deg_sh.at[pl.ds(sid * RPT, RPT)])
        plsc.subcore_barrier()

        def body(j, _):
            pltpu.sync_copy(ones_v, deg_sh.at[dst_v.at[j]], add=True)
            return ()

        lax.fori_loop(0, NB, body, (), unroll=4)
        plsc.subcore_barrier()
        pltpu.sync_copy(
            deg_sh.at[pl.ds(sid * RPT, RPT)],
            deg_out.at[pl.ds(cid * NPAD + sid * RPT, RPT)],
        )

    @functools.partial(
        pl.kernel,
        out_type=jax.ShapeDtypeStruct((NC * NPAD, DH), jnp.float32),
        mesh=mesh,
        compiler_params=params,
        scratch_types=[
            pltpu.VMEM((NB, B), jnp.int32),
            pltpu.VMEM((NB, B), jnp.int32),
            pltpu.VMEM((B, DH), jnp.float32),
            pltpu.VMEM((B, DH), jnp.float32),
            pltpu.VMEM_SHARED((NPAD, DH), jnp.float32),
            pltpu.SemaphoreType.DMA,
            pltpu.SemaphoreType.DMA,
        ],
    )
    def prop_kernel(hs, srcR, dstR, s_out, src_v, dst_v, rows_a, rows_b,
                    agg_sh, sem_a, sem_b):
        cid = lax.axis_index("c")
        sid = lax.axis_index("s")
        wid = sid * NC + cid
        pltpu.sync_copy(srcR.at[pl.ds(wid * NB, NB)], src_v)
        pltpu.sync_copy(dstR.at[pl.ds(wid * NB, NB)], dst_v)
        pltpu.sync_copy(hs.at[pl.ds(sid * RPT, RPT)],
                        agg_sh.at[pl.ds(sid * RPT, RPT)])
        plsc.subcore_barrier()

        cp = pltpu.async_copy(hs.at[src_v.at[0]], rows_a, sem_a)

        def body(i, _):
            j = i * 2
            nxt = pltpu.async_copy(hs.at[src_v.at[j + 1]], rows_b, sem_b)
            cp_wait = pltpu.make_async_copy(hs.at[src_v.at[j]], rows_a, sem_a)
            cp_wait.wait()
            pltpu.sync_copy(rows_a, agg_sh.at[dst_v.at[j]], add=True)

            @pl.when(j + 2 < NB)
            def _():
                pltpu.async_copy(hs.at[src_v.at[j + 2]], rows_a, sem_a)

            nxt_wait = pltpu.make_async_copy(hs.at[src_v.at[j + 1]], rows_b, sem_b)
            nxt_wait.wait()
            pltpu.sync_copy(rows_b, agg_sh.at[dst_v.at[j + 1]], add=True)
            return ()

        lax.fori_loop(0, NB // 2, body, ())
        plsc.subcore_barrier()
        pltpu.sync_copy(
            agg_sh.at[pl.ds(sid * RPT, RPT)],
            s_out.at[pl.ds(cid * NPAD + sid * RPT, RPT)],
        )

    return deg_kernel, prop_kernel


def _pre_body(xb, w0b, b0b, degb, x0b, hsb, Db):
    h = jnp.dot(xb[...], w0b[...], preferred_element_type=jnp.float32)
    h = jnp.maximum(h + b0b[0:1, :], 0.0)
    deg = degb[0, :, :1] + degb[1, :, :1] - 1.0
    D = jnp.broadcast_to(lax.rsqrt(deg), (RB, DH))
    x0b[...] = h
    Db[...] = D
    hsb[...] = D * h


@functools.cache
def _pre_kernel():
    return pl.pallas_call(
        _pre_body,
        grid=(NPAD // RB,),
        in_specs=[
            pl.BlockSpec((RB, D_IN), lambda i: (i, 0)),
            pl.BlockSpec((D_IN, DH), lambda i: (0, 0)),
            pl.BlockSpec((8, DH), lambda i: (0, 0)),
            pl.BlockSpec((2, RB, DEGW), lambda i: (0, i, 0)),
        ],
        out_specs=[
            pl.BlockSpec((RB, DH), lambda i: (i, 0)),
            pl.BlockSpec((RB, DH), lambda i: (i, 0)),
            pl.BlockSpec((RB, DH), lambda i: (i, 0)),
        ],
        out_shape=[jax.ShapeDtypeStruct((NPAD, DH), jnp.float32)] * 3,
    )


def _upd_body(beta, last, sb, hsb, x0b, Db, wcb, w1b, b1b, ob):
    S = sb[0] + sb[1] - hsb[...]
    t = (1.0 - ALPHA) * (Db[...] * S) + ALPHA * x0b[...]
    u = (1.0 - beta) * t + beta * jnp.dot(
        t, wcb[...], preferred_element_type=jnp.float32)
    h = jnp.maximum(u, 0.0)
    if last:
        ob[...] = jnp.dot(h, w1b[...],
                          preferred_element_type=jnp.float32) + b1b[0:1, :]
    else:
        ob[...] = Db[...] * h


@functools.cache
def _upd_kernel(beta, last):
    return pl.pallas_call(
        functools.partial(_upd_body, beta, last),
        grid=(NPAD // RB,),
        in_specs=[
            pl.BlockSpec((2, RB, DH), lambda i: (0, i, 0)),
            pl.BlockSpec((RB, DH), lambda i: (i, 0)),
            pl.BlockSpec((RB, DH), lambda i: (i, 0)),
            pl.BlockSpec((RB, DH), lambda i: (i, 0)),
            pl.BlockSpec((DH, DH), lambda i: (0, 0)),
            pl.BlockSpec((DH, DH), lambda i: (0, 0)),
            pl.BlockSpec((8, DH), lambda i: (0, 0)),
        ],
        out_specs=pl.BlockSpec((RB, DH), lambda i: (i, 0)),
        out_shape=jax.ShapeDtypeStruct((NPAD, DH), jnp.float32),
    )


def kernel(x, edge_index, W0, b0, W1, b1, Wc):
    deg_kernel, prop_kernel = _sc_kernels()

    src = edge_index[0].astype(jnp.int32)
    dst = edge_index[1].astype(jnp.int32)
    srcR = jnp.concatenate(
        [src, jnp.zeros((EPAD - E,), jnp.int32)]).reshape(EPAD // B, B)
    dstR = jnp.concatenate(
        [dst, jnp.full((EPAD - E,), NPAD - 1, jnp.int32)]).reshape(EPAD // B, B)
    xp = jnp.pad(x, ((0, NPAD - N), (0, 0)))
    ones2 = jnp.ones((RPT, DEGW), jnp.float32)
    b0_8 = jnp.tile(b0[None, :], (8, 1))
    b1_8 = jnp.tile(b1[None, :], (8, 1))

    deg = deg_kernel(dstR, ones2).reshape(NC, NPAD, DEGW)
    x0, hs, D = _pre_kernel()(xp, W0, b0_8, deg)
    for l in range(L):
        beta = float(np.log(THETA / (l + 1) + 1.0))
        s = prop_kernel(hs, srcR, dstR).reshape(NC, NPAD, DH)
        hs = _upd_kernel(beta, l == L - 1)(s, hs, x0, D, Wc[l], W1, b1_8)
    return hs[:N]
